# baseline (speedup 1.0000x reference)
import functools

import jax
import jax.numpy as jnp
from jax import lax
from jax.experimental import pallas as pl
from jax.experimental.pallas import tpu as pltpu
from jax.experimental.pallas import tpu_sc as plsc

_TC_BLOCK = 4096
_SC_CHUNK = 64
_LANES = 16


def _tc_z_block(s_ref, x_ref, u_ref, z_ref):
    x = x_ref[...]
    norm = jnp.sqrt(jnp.sum(x * x, axis=1, keepdims=True))
    norm = jnp.maximum(norm, 1e-12)
    y = (s_ref[0] / norm) * x
    z_ref[...] = jnp.dot(y, u_ref[...], preferred_element_type=jnp.float32)


def _tc_z(x, scale, Ulda):
    batch, dim = x.shape
    block = min(_TC_BLOCK, batch)
    return pl.pallas_call(
        _tc_z_block,
        grid=(batch // block,),
        in_specs=[
            pl.BlockSpec(memory_space=pltpu.SMEM),
            pl.BlockSpec((block, dim), lambda i: (i, 0)),
            pl.BlockSpec((dim, dim), lambda i: (0, 0)),
        ],
        out_specs=pl.BlockSpec((block, dim), lambda i: (i, 0)),
        out_shape=jax.ShapeDtypeStruct((batch, dim), jnp.float32),
        compiler_params=pltpu.CompilerParams(
            dimension_semantics=("arbitrary",),
        ),
    )(scale, x, Ulda)


def _make_sc_y(batch, dim):
    info = plsc.get_sparse_core_info()
    nc, ns = info.num_cores, info.num_subcores
    nw = nc * ns
    rows_per_w = batch // nw
    chunk = min(_SC_CHUNK, rows_per_w)
    n_chunks = rows_per_w // chunk
    n_col = dim // _LANES

    mesh = plsc.VectorSubcoreMesh(core_axis_name="c", subcore_axis_name="s")

    @functools.partial(
        pl.kernel,
        mesh=mesh,
        out_type=jax.ShapeDtypeStruct((batch, dim), jnp.float32),
        scratch_types=[
            pltpu.VMEM((chunk, dim), jnp.float32),
            pltpu.VMEM((chunk, dim), jnp.float32),
            pltpu.VMEM((_LANES,), jnp.float32),
        ],
    )
    def sc_y(x_hbm, svec_hbm, y_hbm, xbuf, ybuf, sbuf):
        wid = lax.axis_index("s") * nc + lax.axis_index("c")
        base = wid * rows_per_w
        pltpu.sync_copy(svec_hbm, sbuf)
        svec = sbuf[...]

        def row_body(r, _):
            acc = jnp.zeros((_LANES,), jnp.float32)
            for c in range(n_col):
                xv = xbuf[r, pl.ds(c * _LANES, _LANES)]
                acc = acc + xv * xv
            lane = lax.iota(jnp.int32, _LANES)
            dnums = lax.GatherDimensionNumbers(
                offset_dims=(), collapsed_slice_dims=(0,), start_index_map=(0,)
            )
            for sh in (8, 4, 2, 1):
                perm = jnp.bitwise_xor(lane, sh)[:, None]
                acc = acc + lax.gather(
                    acc,
                    perm,
                    dnums,
                    slice_sizes=(1,),
                    mode=lax.GatherScatterMode.PROMISE_IN_BOUNDS,
                )
            total = jnp.maximum(acc, 1e-24)
            bits = lax.bitcast_convert_type(total, jnp.int32)
            bits = jnp.int32(0x5F3759DF) - lax.shift_right_logical(
                bits, jnp.int32(1)
            )
            g = lax.bitcast_convert_type(bits, jnp.float32)
            half = total * 0.5
            for _unused in range(3):
                g = g * (1.5 - half * g * g)
            rowscale = svec * g
            for c in range(n_col):
                ybuf[r, pl.ds(c * _LANES, _LANES)] = (
                    xbuf[r, pl.ds(c * _LANES, _LANES)] * rowscale
                )
            return _

        for ci in range(n_chunks):
            r0 = base + ci * chunk
            pltpu.sync_copy(x_hbm.at[pl.ds(r0, chunk)], xbuf)
            lax.fori_loop(0, chunk, row_body, 0)
            pltpu.sync_copy(ybuf, y_hbm.at[pl.ds(r0, chunk)])

    return sc_y


def kernel(x, norm_scale, Ulda):
    batch, dim = x.shape
    scale = jnp.reshape(norm_scale.astype(jnp.float32), (1,))
    svec = jnp.broadcast_to(norm_scale.astype(jnp.float32), (_LANES,))
    y = _make_sc_y(batch, dim)(x, svec)
    z = _tc_z(x, scale, Ulda)
    return (y, z)


# speedup vs baseline: 1.2779x; 1.2779x over previous
import functools

import jax
import jax.numpy as jnp
from jax import lax
from jax.experimental import pallas as pl
from jax.experimental.pallas import tpu as pltpu
from jax.experimental.pallas import tpu_sc as plsc

_TC_BLOCK = 4096
_SC_CHUNK = 64
_LANES = 16


def _tc_z_block(s_ref, x_ref, u_ref, z_ref):
    x = x_ref[...]
    norm = jnp.sqrt(jnp.sum(x * x, axis=1, keepdims=True))
    norm = jnp.maximum(norm, 1e-12)
    y = (s_ref[0] / norm) * x
    z_ref[...] = jnp.dot(y, u_ref[...], preferred_element_type=jnp.float32)


def _tc_z(x, scale, Ulda):
    batch, dim = x.shape
    block = min(_TC_BLOCK, batch)
    return pl.pallas_call(
        _tc_z_block,
        grid=(batch // block,),
        in_specs=[
            pl.BlockSpec(memory_space=pltpu.SMEM),
            pl.BlockSpec((block, dim), lambda i: (i, 0)),
            pl.BlockSpec((dim, dim), lambda i: (0, 0)),
        ],
        out_specs=pl.BlockSpec((block, dim), lambda i: (i, 0)),
        out_shape=jax.ShapeDtypeStruct((batch, dim), jnp.float32),
        compiler_params=pltpu.CompilerParams(
            dimension_semantics=("arbitrary",),
        ),
    )(scale, x, Ulda)


def _make_sc_y(batch, dim):
    info = plsc.get_sparse_core_info()
    nc, ns = info.num_cores, info.num_subcores
    nw = nc * ns
    rows_per_w = batch // nw
    chunk = min(_SC_CHUNK, rows_per_w)
    n_chunks = rows_per_w // chunk
    n_col = dim // _LANES

    mesh = plsc.VectorSubcoreMesh(core_axis_name="c", subcore_axis_name="s")

    @functools.partial(
        pl.kernel,
        mesh=mesh,
        out_type=jax.ShapeDtypeStruct((batch, dim), jnp.float32),
        scratch_types=[
            pltpu.VMEM((2, chunk, dim), jnp.float32),
            pltpu.VMEM((2, chunk, dim), jnp.float32),
            pltpu.VMEM((_LANES,), jnp.float32),
            pltpu.SemaphoreType.DMA,
            pltpu.SemaphoreType.DMA,
            pltpu.SemaphoreType.DMA,
            pltpu.SemaphoreType.DMA,
        ],
    )
    def sc_y(x_hbm, svec_hbm, y_hbm, xbuf, ybuf, sbuf, si0, si1, so0, so1):
        wid = lax.axis_index("s") * nc + lax.axis_index("c")
        base = wid * rows_per_w
        pltpu.sync_copy(svec_hbm, sbuf)
        svec = sbuf[...]
        sin = (si0, si1)
        sout = (so0, so1)

        def make_row_body(xb, yb):
            def row_body(r, carry):
                xs = []
                acc = None
                for c in range(n_col):
                    xv = xb[r, pl.ds(c * _LANES, _LANES)]
                    xs.append(xv)
                    acc = xv * xv if acc is None else acc + xv * xv
                lane = lax.iota(jnp.int32, _LANES)
                dnums = lax.GatherDimensionNumbers(
                    offset_dims=(),
                    collapsed_slice_dims=(0,),
                    start_index_map=(0,),
                )
                for sh in (8, 4, 2, 1):
                    perm = jnp.bitwise_xor(lane, sh)[:, None]
                    acc = acc + lax.gather(
                        acc,
                        perm,
                        dnums,
                        slice_sizes=(1,),
                        mode=lax.GatherScatterMode.PROMISE_IN_BOUNDS,
                    )
                total = jnp.maximum(acc, 1e-24)
                bits = lax.bitcast_convert_type(total, jnp.int32)
                bits = jnp.int32(0x5F3759DF) - lax.shift_right_logical(
                    bits, jnp.int32(1)
                )
                g = lax.bitcast_convert_type(bits, jnp.float32)
                half = total * 0.5
                for _unused in range(3):
                    g = g * (1.5 - half * g * g)
                rowscale = svec * g
                for c in range(n_col):
                    yb[r, pl.ds(c * _LANES, _LANES)] = xs[c] * rowscale
                return carry

            return row_body

        hin = [None, None]
        hout = [None, None]
        hin[0] = pltpu.async_copy(
            x_hbm.at[pl.ds(base, chunk)], xbuf.at[0], sin[0]
        )
        for ci in range(n_chunks):
            b = ci % 2
            if ci + 1 < n_chunks:
                nb = (ci + 1) % 2
                hin[nb] = pltpu.async_copy(
                    x_hbm.at[pl.ds(base + (ci + 1) * chunk, chunk)],
                    xbuf.at[nb],
                    sin[nb],
                )
            hin[b].wait()
            if ci >= 2:
                hout[b].wait()
            lax.fori_loop(
                0, chunk, make_row_body(xbuf.at[b], ybuf.at[b]), 0
            )
            hout[b] = pltpu.async_copy(
                ybuf.at[b], y_hbm.at[pl.ds(base + ci * chunk, chunk)], sout[b]
            )
        hout[(n_chunks - 1) % 2].wait()
        if n_chunks >= 2:
            hout[n_chunks % 2].wait()

    return sc_y


def kernel(x, norm_scale, Ulda):
    batch, dim = x.shape
    scale = jnp.reshape(norm_scale.astype(jnp.float32), (1,))
    svec = jnp.broadcast_to(norm_scale.astype(jnp.float32), (_LANES,))
    y = _make_sc_y(batch, dim)(x, svec)
    z = _tc_z(x, scale, Ulda)
    return (y, z)


# speedup vs baseline: 2.9725x; 2.3261x over previous
import jax
import jax.numpy as jnp
from jax import lax
from jax.experimental import pallas as pl
from jax.experimental.pallas import tpu as pltpu

_BLK = 1024
_NB = 4


def _plda_manual(s_ref, x_hbm, u_ref, y_hbm, z_hbm, xb, yb, zb, si, sy, sz):
    nblk = x_hbm.shape[0] // _BLK
    s = s_ref[0]
    u = u_ref[...]

    def load(i, j):
        return pltpu.make_async_copy(
            x_hbm.at[pl.ds(i * _BLK, _BLK)], xb.at[j], si.at[j]
        )

    def store_y(i, j):
        return pltpu.make_async_copy(
            yb.at[j], y_hbm.at[pl.ds(i * _BLK, _BLK)], sy.at[j]
        )

    def store_z(i, j):
        return pltpu.make_async_copy(
            zb.at[j], z_hbm.at[pl.ds(i * _BLK, _BLK)], sz.at[j]
        )

    for j in range(_NB):
        load(j, j).start()

    def body(i, carry):
        j = lax.rem(i, _NB)
        load(i, j).wait()

        @pl.when(i >= _NB)
        def _():
            store_y(i - _NB, j).wait()
            store_z(i - _NB, j).wait()

        x = xb[j]
        norm = jnp.sqrt(jnp.sum(x * x, axis=1, keepdims=True))
        norm = jnp.maximum(norm, 1e-12)
        y = (s / norm) * x
        yb[j] = y
        zb[j] = jnp.dot(y, u, preferred_element_type=jnp.float32)
        store_y(i, j).start()
        store_z(i, j).start()

        @pl.when(i + _NB < nblk)
        def _():
            load(i + _NB, j).start()

        return carry

    lax.fori_loop(0, nblk, body, 0)
    for i in range(nblk - _NB, nblk):
        j = i % _NB
        store_y(i, j).wait()
        store_z(i, j).wait()


def kernel(x, norm_scale, Ulda):
    batch, dim = x.shape
    scale = jnp.reshape(norm_scale.astype(jnp.float32), (1,))
    y, z = pl.pallas_call(
        _plda_manual,
        in_specs=[
            pl.BlockSpec(memory_space=pltpu.SMEM),
            pl.BlockSpec(memory_space=pl.ANY),
            pl.BlockSpec(memory_space=pltpu.VMEM),
        ],
        out_specs=[
            pl.BlockSpec(memory_space=pl.ANY),
            pl.BlockSpec(memory_space=pl.ANY),
        ],
        out_shape=[
            jax.ShapeDtypeStruct((batch, dim), jnp.float32),
            jax.ShapeDtypeStruct((batch, dim), jnp.float32),
        ],
        scratch_shapes=[
            pltpu.VMEM((_NB, _BLK, dim), jnp.float32),
            pltpu.VMEM((_NB, _BLK, dim), jnp.float32),
            pltpu.VMEM((_NB, _BLK, dim), jnp.float32),
            pltpu.SemaphoreType.DMA((_NB,)),
            pltpu.SemaphoreType.DMA((_NB,)),
            pltpu.SemaphoreType.DMA((_NB,)),
        ],
    )(scale, x, Ulda)
    return (y, z)


# speedup vs baseline: 3.0309x; 1.0197x over previous
import jax
import jax.numpy as jnp
from jax import lax
from jax.experimental import pallas as pl
from jax.experimental.pallas import tpu as pltpu

_BLK = 2048
_NB = 3


def _plda_manual(s_ref, x_hbm, u_ref, y_hbm, z_hbm, xb, yb, zb, si, sy, sz):
    nblk = x_hbm.shape[0] // _BLK
    s = s_ref[0]
    u = u_ref[...]

    def load(i, j):
        return pltpu.make_async_copy(
            x_hbm.at[pl.ds(i * _BLK, _BLK)], xb.at[j], si.at[j]
        )

    def store_y(i, j):
        return pltpu.make_async_copy(
            yb.at[j], y_hbm.at[pl.ds(i * _BLK, _BLK)], sy.at[j]
        )

    def store_z(i, j):
        return pltpu.make_async_copy(
            zb.at[j], z_hbm.at[pl.ds(i * _BLK, _BLK)], sz.at[j]
        )

    for j in range(_NB):
        load(j, j).start()

    def body(i, carry):
        j = lax.rem(i, _NB)
        load(i, j).wait()

        @pl.when(i >= _NB)
        def _():
            store_y(i - _NB, j).wait()
            store_z(i - _NB, j).wait()

        x = xb[j]
        norm = jnp.sqrt(jnp.sum(x * x, axis=1, keepdims=True))
        norm = jnp.maximum(norm, 1e-12)
        y = (s / norm) * x
        yb[j] = y
        zb[j] = jnp.dot(y, u, preferred_element_type=jnp.float32)
        store_y(i, j).start()
        store_z(i, j).start()

        @pl.when(i + _NB < nblk)
        def _():
            load(i + _NB, j).start()

        return carry

    lax.fori_loop(0, nblk, body, 0)
    for i in range(nblk - _NB, nblk):
        j = i % _NB
        store_y(i, j).wait()
        store_z(i, j).wait()


def kernel(x, norm_scale, Ulda):
    batch, dim = x.shape
    scale = jnp.reshape(norm_scale.astype(jnp.float32), (1,))
    y, z = pl.pallas_call(
        _plda_manual,
        in_specs=[
            pl.BlockSpec(memory_space=pltpu.SMEM),
            pl.BlockSpec(memory_space=pl.ANY),
            pl.BlockSpec(memory_space=pltpu.VMEM),
        ],
        out_specs=[
            pl.BlockSpec(memory_space=pl.ANY),
            pl.BlockSpec(memory_space=pl.ANY),
        ],
        out_shape=[
            jax.ShapeDtypeStruct((batch, dim), jnp.float32),
            jax.ShapeDtypeStruct((batch, dim), jnp.float32),
        ],
        scratch_shapes=[
            pltpu.VMEM((_NB, _BLK, dim), jnp.float32),
            pltpu.VMEM((_NB, _BLK, dim), jnp.float32),
            pltpu.VMEM((_NB, _BLK, dim), jnp.float32),
            pltpu.SemaphoreType.DMA((_NB,)),
            pltpu.SemaphoreType.DMA((_NB,)),
            pltpu.SemaphoreType.DMA((_NB,)),
        ],
    )(scale, x, Ulda)
    return (y, z)


# speedup vs baseline: 3.1532x; 1.0403x over previous
import jax
import jax.numpy as jnp
from jax import lax
from jax.experimental import pallas as pl
from jax.experimental.pallas import tpu as pltpu

_BLK = 4096
_NB = 3


def _plda_manual(s_ref, x_hbm, u_ref, y_hbm, z_hbm, xb, yb, zb, si, sy, sz):
    nblk = x_hbm.shape[0] // _BLK
    s = s_ref[0]
    u = u_ref[...]

    def load(i, j):
        return pltpu.make_async_copy(
            x_hbm.at[pl.ds(i * _BLK, _BLK)], xb.at[j], si.at[j]
        )

    def store_y(i, j):
        return pltpu.make_async_copy(
            yb.at[j], y_hbm.at[pl.ds(i * _BLK, _BLK)], sy.at[j]
        )

    def store_z(i, j):
        return pltpu.make_async_copy(
            zb.at[j], z_hbm.at[pl.ds(i * _BLK, _BLK)], sz.at[j]
        )

    for j in range(_NB):
        load(j, j).start()

    def body(i, carry):
        j = lax.rem(i, _NB)
        load(i, j).wait()

        @pl.when(i >= _NB)
        def _():
            store_y(i - _NB, j).wait()
            store_z(i - _NB, j).wait()

        x = xb[j]
        norm = jnp.sqrt(jnp.sum(x * x, axis=1, keepdims=True))
        norm = jnp.maximum(norm, 1e-12)
        y = (s / norm) * x
        yb[j] = y
        zb[j] = jnp.dot(y, u, preferred_element_type=jnp.float32)
        store_y(i, j).start()
        store_z(i, j).start()

        @pl.when(i + _NB < nblk)
        def _():
            load(i + _NB, j).start()

        return carry

    lax.fori_loop(0, nblk, body, 0)
    for i in range(nblk - _NB, nblk):
        j = i % _NB
        store_y(i, j).wait()
        store_z(i, j).wait()


def kernel(x, norm_scale, Ulda):
    batch, dim = x.shape
    scale = jnp.reshape(norm_scale.astype(jnp.float32), (1,))
    y, z = pl.pallas_call(
        _plda_manual,
        in_specs=[
            pl.BlockSpec(memory_space=pltpu.SMEM),
            pl.BlockSpec(memory_space=pl.ANY),
            pl.BlockSpec(memory_space=pltpu.VMEM),
        ],
        out_specs=[
            pl.BlockSpec(memory_space=pl.ANY),
            pl.BlockSpec(memory_space=pl.ANY),
        ],
        out_shape=[
            jax.ShapeDtypeStruct((batch, dim), jnp.float32),
            jax.ShapeDtypeStruct((batch, dim), jnp.float32),
        ],
        scratch_shapes=[
            pltpu.VMEM((_NB, _BLK, dim), jnp.float32),
            pltpu.VMEM((_NB, _BLK, dim), jnp.float32),
            pltpu.VMEM((_NB, _BLK, dim), jnp.float32),
            pltpu.SemaphoreType.DMA((_NB,)),
            pltpu.SemaphoreType.DMA((_NB,)),
            pltpu.SemaphoreType.DMA((_NB,)),
        ],
    )(scale, x, Ulda)
    return (y, z)
